# Initial kernel scaffold; baseline (speedup 1.0000x reference)
import jax
import jax.numpy as jnp
from jax.experimental import pallas as pl


def kernel(input_atom, input_bond, atom_graph, bond_graph, num_nbs, node_mask, W_af, b_af, W_nb, b_nb, W_na, b_na, W_sa, b_sa, W_u2, b_u2, W_u1, b_u1, Wq, bq, Wk, bk, Wv, bv, Wam, bam, Wg, bg):
    raise NotImplementedError("write your pallas kernel here")


# speedup vs baseline: 1.3734x; 1.3734x over previous
import functools

import jax
import jax.numpy as jnp
from jax import lax
from jax.experimental import pallas as pl
from jax.experimental.pallas import tpu as pltpu

B = 4
N = 4096
HID = 128
MAXNEI = 10
DEPTH = 3
E = N * MAXNEI

T0 = 512
T1 = 64
T2 = 512

F32 = jnp.float32


def _dot(x, w):
    return jnp.dot(x, w, preferred_element_type=F32)


def _pre_body(ia_ref, ib_ref, waf_ref, baf_ref, wau_ref, bau_ref, wbt_ref, bbt_ref,
              af_ref, atab_ref, btab_ref):
    ia = ia_ref[0]
    ib = ib_ref[0]
    af = _dot(ia, waf_ref[...]) + baf_ref[...]
    af_ref[0] = af
    atab_ref[0] = _dot(af, wau_ref[...]) + bau_ref[...]
    btab_ref[0] = _dot(ib, wbt_ref[...]) + bbt_ref[...]


def _run_pre(input_atom, input_bond, waf, baf, wau, bau, wbt, bbt):
    grid = (B, N // T0)
    full = lambda shp: pl.BlockSpec(shp, lambda b, t: tuple(0 for _ in shp))
    return pl.pallas_call(
        _pre_body,
        grid=grid,
        in_specs=[
            pl.BlockSpec((1, T0, 34), lambda b, t: (b, t, 0)),
            pl.BlockSpec((1, T0, 46), lambda b, t: (b, t, 0)),
            full((34, HID)), full((1, HID)),
            full((HID, 256)), full((1, 256)),
            full((46, 256)), full((1, 256)),
        ],
        out_specs=[
            pl.BlockSpec((1, T0, HID), lambda b, t: (b, t, 0)),
            pl.BlockSpec((1, T0, 256), lambda b, t: (b, t, 0)),
            pl.BlockSpec((1, T0, 256), lambda b, t: (b, t, 0)),
        ],
        out_shape=[
            jax.ShapeDtypeStruct((B, N, HID), F32),
            jax.ShapeDtypeStruct((B, N, 256), F32),
            jax.ShapeDtypeStruct((B, N, 256), F32),
        ],
    )(input_atom, input_bond, waf, baf, wau, bau, wbt, bbt)


def _k1_body(af_ref, ag_ref, bgt_ref, m_ref, nm_ref, lin_ref,
             wsa_ref, bsa_ref, wk_ref, bk_ref, wv_ref, bv_ref,
             wam_ref, bam_ref, wq_ref, bq_ref, wg_ref, bg2_ref,
             lout_ref, att_ref, gl_ref):
    T = T1
    af = af_ref[0]
    AG = ag_ref[0].reshape(T, MAXNEI, 256)
    BGt = bgt_ref[0].reshape(T, MAXNEI, 256)
    msk = m_ref[0].reshape(T, MAXNEI, 1)
    h_nei_atom = AG[:, :, :HID]
    u2a = AG[:, :, HID:]
    h_nei_bond = BGt[:, :, :HID]
    bt_u2 = BGt[:, :, HID:]
    f_nei = (h_nei_atom * h_nei_bond * msk).sum(axis=1)
    f_self = _dot(af, wsa_ref[...]) + bsa_ref[...]
    lout_ref[0] = lin_ref[0] + f_nei * f_self * nm_ref[0] * (1.0 / DEPTH)

    pre_label = jnp.maximum(u2a + bt_u2, 0.0)
    nei_label = pre_label * msk
    nei_att = jnp.concatenate([nei_label, h_nei_bond], axis=2)
    na2 = nei_att.reshape(T * MAXNEI, 256)
    k = _dot(na2, wk_ref[...]) + bk_ref[...]
    v = _dot(na2, wv_ref[...]) + bv_ref[...]
    am = _dot(na2, wam_ref[...]) + bam_ref[...]
    q = _dot(af, wq_ref[...]) + bq_ref[...]

    k3 = k.reshape(T, MAXNEI, HID)
    v3 = v.reshape(T, MAXNEI, HID)
    am3 = am.reshape(T, MAXNEI, HID)
    wg = wg_ref[...]
    att_parts = []
    gl_parts = []
    for j in range(4):
        qj = q[:, 32 * j:32 * (j + 1)]
        sj = []
        for mp in range(MAXNEI):
            t40 = 10 * j + mp
            kc = k3[:, t40 // 4, 32 * (t40 % 4):32 * (t40 % 4) + 32]
            sj.append((qj * kc).sum(axis=-1, keepdims=True))
        sjm = jnp.concatenate(sj, axis=-1)
        sjm = sjm - sjm.max(axis=-1, keepdims=True)
        es = jnp.exp(sjm)
        sm = es / es.sum(axis=-1, keepdims=True)
        att_j = jnp.zeros((T, 32), F32)
        emax_j = None
        for mp in range(MAXNEI):
            t40 = 10 * j + mp
            c0 = 32 * (t40 % 4)
            vc = v3[:, t40 // 4, c0:c0 + 32]
            ac = am3[:, t40 // 4, c0:c0 + 32]
            att_j = att_j + sm[:, mp:mp + 1] * vc
            emax_j = ac if emax_j is None else jnp.maximum(emax_j, ac)
        ave_j = jnp.zeros((T, 64), F32)
        for mp in range(MAXNEI):
            p0 = 640 * j + 64 * mp
            ave_j = ave_j + nei_att[:, p0 // 256, p0 % 256:p0 % 256 + 64]
        ave_j = ave_j * (1.0 / MAXNEI)
        glj = ((qjc := af[:, 32 * j:32 * (j + 1)]) * wg[:, 0:32]).sum(-1, keepdims=True)
        glj = glj + (emax_j * wg[:, 32:64]).sum(-1, keepdims=True)
        glj = glj + (ave_j * wg[:, 64:HID]).sum(-1, keepdims=True)
        att_parts.append(att_j)
        gl_parts.append(glj)
    att_ref[0] = jnp.concatenate(att_parts, axis=-1)
    gl_ref[0] = jnp.concatenate(gl_parts, axis=-1) + bg2_ref[0, 0]


def _run_k1(af, AG, BGt, mask, nm, lin, wsa, bsa, wk, bk, wv, bv, wam, bam,
            wq, bq, wg, bg2):
    grid = (B, N // T1)
    full = lambda shp: pl.BlockSpec(shp, lambda b, t: tuple(0 for _ in shp))
    return pl.pallas_call(
        _k1_body,
        grid=grid,
        in_specs=[
            pl.BlockSpec((1, T1, HID), lambda b, t: (b, t, 0)),
            pl.BlockSpec((1, T1 * MAXNEI, 256), lambda b, t: (b, t, 0)),
            pl.BlockSpec((1, T1 * MAXNEI, 256), lambda b, t: (b, t, 0)),
            pl.BlockSpec((1, T1, MAXNEI), lambda b, t: (b, t, 0)),
            pl.BlockSpec((1, T1, 1), lambda b, t: (b, t, 0)),
            pl.BlockSpec((1, T1, HID), lambda b, t: (b, t, 0)),
            full((HID, HID)), full((1, HID)),
            full((256, HID)), full((1, HID)),
            full((256, HID)), full((1, HID)),
            full((256, HID)), full((1, HID)),
            full((HID, HID)), full((1, HID)),
            full((1, HID)), full((1, 1)),
        ],
        out_specs=[
            pl.BlockSpec((1, T1, HID), lambda b, t: (b, t, 0)),
            pl.BlockSpec((1, T1, HID), lambda b, t: (b, t, 0)),
            pl.BlockSpec((1, T1, 4), lambda b, t: (b, t, 0)),
        ],
        out_shape=[
            jax.ShapeDtypeStruct((B, N, HID), F32),
            jax.ShapeDtypeStruct((B, N, HID), F32),
            jax.ShapeDtypeStruct((B, N, 4), F32),
        ],
    )(af, AG, BGt, mask, nm, lin, wsa, bsa, wk, bk, wv, bv, wam, bam, wq, bq, wg, bg2)


def _k2_body(att_ref, gl_ref, af_ref, wu1_ref, bu1_ref, wau_ref, bau_ref,
             afn_ref, atab_ref):
    att = att_ref[0]
    gl = gl_ref[0]
    af = af_ref[0]
    m = gl.max(axis=0, keepdims=True)
    eg = jnp.exp(gl - m)
    g = eg / eg.sum(axis=0, keepdims=True)
    f_att = jnp.concatenate(
        [att[:, :, 32 * j:32 * (j + 1)] * g[:, :, j:j + 1] for j in range(4)],
        axis=-1)
    nl = jnp.concatenate([af, f_att], axis=-1).reshape(4 * T2, 256)
    afn = jnp.maximum(_dot(nl, wu1_ref[...]) + bu1_ref[...], 0.0)
    afn_ref[0] = afn.reshape(4, T2, HID)
    atab_ref[0] = (_dot(afn, wau_ref[...]) + bau_ref[...]).reshape(4, T2, 256)


def _run_k2(att4, gl4, af4, wu1, bu1, wau, bau):
    grid = (B, 1024 // T2)
    full = lambda shp: pl.BlockSpec(shp, lambda b, t: tuple(0 for _ in shp))
    return pl.pallas_call(
        _k2_body,
        grid=grid,
        in_specs=[
            pl.BlockSpec((1, 4, T2, HID), lambda b, t: (b, 0, t, 0)),
            pl.BlockSpec((1, 4, T2, 4), lambda b, t: (b, 0, t, 0)),
            pl.BlockSpec((1, 4, T2, HID), lambda b, t: (b, 0, t, 0)),
            full((256, HID)), full((1, HID)),
            full((HID, 256)), full((1, 256)),
        ],
        out_specs=[
            pl.BlockSpec((1, 4, T2, HID), lambda b, t: (b, 0, t, 0)),
            pl.BlockSpec((1, 4, T2, 256), lambda b, t: (b, 0, t, 0)),
        ],
        out_shape=[
            jax.ShapeDtypeStruct((B, 4, 1024, HID), F32),
            jax.ShapeDtypeStruct((B, 4, 1024, 256), F32),
        ],
    )(att4, gl4, af4, wu1, bu1, wau, bau)


def _gather_rows(tab_flat, gidx):
    return jnp.take(tab_flat, gidx, axis=0)


def kernel(input_atom, input_bond, atom_graph, bond_graph, num_nbs, node_mask,
           W_af, b_af, W_nb, b_nb, W_na, b_na, W_sa, b_sa, W_u2, b_u2, W_u1, b_u1,
           Wq, bq, Wk, bk, Wv, bv, Wam, bam, Wg, bg):
    W_u2a = W_u2[:, :HID]
    W_u2b = W_u2[:, HID:]
    waf = W_af.T
    baf = b_af.reshape(1, HID)
    wau = jnp.concatenate([W_na.T, W_u2a.T], axis=1)
    bau = jnp.concatenate([b_na, b_u2]).reshape(1, 256)
    wbt = jnp.concatenate([W_nb.T, W_u2b.T], axis=1)
    bbt = jnp.concatenate([b_nb, jnp.zeros_like(b_u2)]).reshape(1, 256)
    wsa = W_sa.T
    bsa = b_sa.reshape(1, HID)
    wk = Wk.T
    bk2 = bk.reshape(1, HID)
    wv = Wv.T
    bv2 = bv.reshape(1, HID)
    wam = Wam.T
    bam2 = bam.reshape(1, HID)
    wq = Wq.T
    bq2 = bq.reshape(1, HID)
    wg = Wg.reshape(1, HID)
    bg2 = bg.reshape(1, 1)
    wu1 = W_u1.T
    bu1 = b_u1.reshape(1, HID)

    offs = (jnp.arange(B, dtype=jnp.int32) * N)[:, None, None]
    aidx = (atom_graph[..., 1].astype(jnp.int32) + offs).reshape(B * E)
    bidx = (bond_graph[..., 1].astype(jnp.int32) + offs).reshape(B * E)
    mask = (jnp.arange(MAXNEI, dtype=jnp.int32)[None, None, :]
            < num_nbs[:, :, None].astype(jnp.int32)).astype(F32)
    nm = node_mask.reshape(B, N, 1)

    af, A_tab, B_tab = _run_pre(input_atom, input_bond, waf, baf, wau, bau, wbt, bbt)
    BGt = _gather_rows(B_tab.reshape(B * N, 256), bidx).reshape(B, E, 256)

    lin = jnp.zeros((B, N, HID), F32)
    for d in range(DEPTH):
        AG = _gather_rows(A_tab.reshape(B * N, 256), aidx).reshape(B, E, 256)
        lin, att, gl = _run_k1(af, AG, BGt, mask, nm, lin,
                               wsa, bsa, wk, bk2, wv, bv2, wam, bam2,
                               wq, bq2, wg, bg2)
        if d < DEPTH - 1:
            af4, atab4 = _run_k2(att.reshape(B, 4, 1024, HID),
                                 gl.reshape(B, 4, 1024, 4),
                                 af.reshape(B, 4, 1024, HID),
                                 wu1, bu1, wau, bau)
            af = af4.reshape(B, N, HID)
            A_tab = atab4.reshape(B, N, 256)
    return lin


# speedup vs baseline: 1.7449x; 1.2705x over previous
import functools

import jax
import jax.numpy as jnp
from jax import lax
from jax.experimental import pallas as pl
from jax.experimental.pallas import tpu as pltpu
from jax.experimental.pallas import tpu_sc as plsc

B = 4
N = 4096
HID = 128
MAXNEI = 10
DEPTH = 3
E = N * MAXNEI

T0 = 512
T1 = 64
T2 = 512

F32 = jnp.float32


def _dot(x, w):
    return jnp.dot(x, w, preferred_element_type=F32)


def _pre_body(ia_ref, ib_ref, waf_ref, baf_ref, wau_ref, bau_ref, wbt_ref, bbt_ref,
              af_ref, atab_ref, btab_ref):
    ia = ia_ref[0]
    ib = ib_ref[0]
    af = _dot(ia, waf_ref[...]) + baf_ref[...]
    af_ref[0] = af
    atab_ref[0] = _dot(af, wau_ref[...]) + bau_ref[...]
    btab_ref[0] = _dot(ib, wbt_ref[...]) + bbt_ref[...]


def _run_pre(input_atom, input_bond, waf, baf, wau, bau, wbt, bbt):
    grid = (B, N // T0)
    full = lambda shp: pl.BlockSpec(shp, lambda b, t: tuple(0 for _ in shp))
    return pl.pallas_call(
        _pre_body,
        grid=grid,
        in_specs=[
            pl.BlockSpec((1, T0, 34), lambda b, t: (b, t, 0)),
            pl.BlockSpec((1, T0, 46), lambda b, t: (b, t, 0)),
            full((34, HID)), full((1, HID)),
            full((HID, 256)), full((1, 256)),
            full((46, 256)), full((1, 256)),
        ],
        out_specs=[
            pl.BlockSpec((1, T0, HID), lambda b, t: (b, t, 0)),
            pl.BlockSpec((1, T0, 256), lambda b, t: (b, t, 0)),
            pl.BlockSpec((1, T0, 256), lambda b, t: (b, t, 0)),
        ],
        out_shape=[
            jax.ShapeDtypeStruct((B, N, HID), F32),
            jax.ShapeDtypeStruct((B, N, 256), F32),
            jax.ShapeDtypeStruct((B, N, 256), F32),
        ],
    )(input_atom, input_bond, waf, baf, wau, bau, wbt, bbt)


def _k1_body(af_ref, ag_ref, bgt_ref, m_ref, nm_ref, lin_ref,
             wsa_ref, bsa_ref, wk_ref, bk_ref, wv_ref, bv_ref,
             wam_ref, bam_ref, wq_ref, bq_ref, wg_ref, bg2_ref,
             lout_ref, att_ref, gl_ref):
    T = T1
    af = af_ref[0]
    AG = ag_ref[0].reshape(T, MAXNEI, 256)
    BGt = bgt_ref[0].reshape(T, MAXNEI, 256)
    msk = m_ref[0].reshape(T, MAXNEI, 1)
    h_nei_atom = AG[:, :, :HID]
    u2a = AG[:, :, HID:]
    h_nei_bond = BGt[:, :, :HID]
    bt_u2 = BGt[:, :, HID:]
    f_nei = (h_nei_atom * h_nei_bond * msk).sum(axis=1)
    f_self = _dot(af, wsa_ref[...]) + bsa_ref[...]
    lout_ref[0] = lin_ref[0] + f_nei * f_self * nm_ref[0] * (1.0 / DEPTH)

    pre_label = jnp.maximum(u2a + bt_u2, 0.0)
    nei_label = pre_label * msk
    nei_att = jnp.concatenate([nei_label, h_nei_bond], axis=2)
    na2 = nei_att.reshape(T * MAXNEI, 256)
    k = _dot(na2, wk_ref[...]) + bk_ref[...]
    v = _dot(na2, wv_ref[...]) + bv_ref[...]
    am = _dot(na2, wam_ref[...]) + bam_ref[...]
    q = _dot(af, wq_ref[...]) + bq_ref[...]

    k3 = k.reshape(T, MAXNEI, HID)
    v3 = v.reshape(T, MAXNEI, HID)
    am3 = am.reshape(T, MAXNEI, HID)
    wg = wg_ref[...]
    att_parts = []
    gl_parts = []
    for j in range(4):
        qj = q[:, 32 * j:32 * (j + 1)]
        sj = []
        for mp in range(MAXNEI):
            t40 = 10 * j + mp
            kc = k3[:, t40 // 4, 32 * (t40 % 4):32 * (t40 % 4) + 32]
            sj.append((qj * kc).sum(axis=-1, keepdims=True))
        sjm = jnp.concatenate(sj, axis=-1)
        sjm = sjm - sjm.max(axis=-1, keepdims=True)
        es = jnp.exp(sjm)
        sm = es / es.sum(axis=-1, keepdims=True)
        att_j = jnp.zeros((T, 32), F32)
        emax_j = None
        for mp in range(MAXNEI):
            t40 = 10 * j + mp
            c0 = 32 * (t40 % 4)
            vc = v3[:, t40 // 4, c0:c0 + 32]
            ac = am3[:, t40 // 4, c0:c0 + 32]
            att_j = att_j + sm[:, mp:mp + 1] * vc
            emax_j = ac if emax_j is None else jnp.maximum(emax_j, ac)
        ave_j = jnp.zeros((T, 64), F32)
        for mp in range(MAXNEI):
            p0 = 640 * j + 64 * mp
            ave_j = ave_j + nei_att[:, p0 // 256, p0 % 256:p0 % 256 + 64]
        ave_j = ave_j * (1.0 / MAXNEI)
        glj = ((qjc := af[:, 32 * j:32 * (j + 1)]) * wg[:, 0:32]).sum(-1, keepdims=True)
        glj = glj + (emax_j * wg[:, 32:64]).sum(-1, keepdims=True)
        glj = glj + (ave_j * wg[:, 64:HID]).sum(-1, keepdims=True)
        att_parts.append(att_j)
        gl_parts.append(glj)
    att_ref[0] = jnp.concatenate(att_parts, axis=-1)
    gl_ref[0] = jnp.concatenate(gl_parts, axis=-1) + bg2_ref[0, 0]


def _run_k1(af, AG, BGt, mask, nm, lin, wsa, bsa, wk, bk, wv, bv, wam, bam,
            wq, bq, wg, bg2):
    grid = (B, N // T1)
    full = lambda shp: pl.BlockSpec(shp, lambda b, t: tuple(0 for _ in shp))
    return pl.pallas_call(
        _k1_body,
        grid=grid,
        in_specs=[
            pl.BlockSpec((1, T1, HID), lambda b, t: (b, t, 0)),
            pl.BlockSpec((1, T1 * MAXNEI, 256), lambda b, t: (b, t, 0)),
            pl.BlockSpec((1, T1 * MAXNEI, 256), lambda b, t: (b, t, 0)),
            pl.BlockSpec((1, T1, MAXNEI), lambda b, t: (b, t, 0)),
            pl.BlockSpec((1, T1, 1), lambda b, t: (b, t, 0)),
            pl.BlockSpec((1, T1, HID), lambda b, t: (b, t, 0)),
            full((HID, HID)), full((1, HID)),
            full((256, HID)), full((1, HID)),
            full((256, HID)), full((1, HID)),
            full((256, HID)), full((1, HID)),
            full((HID, HID)), full((1, HID)),
            full((1, HID)), full((1, 1)),
        ],
        out_specs=[
            pl.BlockSpec((1, T1, HID), lambda b, t: (b, t, 0)),
            pl.BlockSpec((1, T1, HID), lambda b, t: (b, t, 0)),
            pl.BlockSpec((1, T1, 4), lambda b, t: (b, t, 0)),
        ],
        out_shape=[
            jax.ShapeDtypeStruct((B, N, HID), F32),
            jax.ShapeDtypeStruct((B, N, HID), F32),
            jax.ShapeDtypeStruct((B, N, 4), F32),
        ],
    )(af, AG, BGt, mask, nm, lin, wsa, bsa, wk, bk, wv, bv, wam, bam, wq, bq, wg, bg2)


def _k2_body(att_ref, gl_ref, af_ref, wu1_ref, bu1_ref, wau_ref, bau_ref,
             afn_ref, atab_ref):
    att = att_ref[0]
    gl = gl_ref[0]
    af = af_ref[0]
    m = gl.max(axis=0, keepdims=True)
    eg = jnp.exp(gl - m)
    g = eg / eg.sum(axis=0, keepdims=True)
    f_att = jnp.concatenate(
        [att[:, :, 32 * j:32 * (j + 1)] * g[:, :, j:j + 1] for j in range(4)],
        axis=-1)
    nl = jnp.concatenate([af, f_att], axis=-1).reshape(4 * T2, 256)
    afn = jnp.maximum(_dot(nl, wu1_ref[...]) + bu1_ref[...], 0.0)
    afn_ref[0] = afn.reshape(4, T2, HID)
    atab_ref[0] = (_dot(afn, wau_ref[...]) + bau_ref[...]).reshape(4, T2, 256)


def _run_k2(att4, gl4, af4, wu1, bu1, wau, bau):
    grid = (B, 1024 // T2)
    full = lambda shp: pl.BlockSpec(shp, lambda b, t: tuple(0 for _ in shp))
    return pl.pallas_call(
        _k2_body,
        grid=grid,
        in_specs=[
            pl.BlockSpec((1, 4, T2, HID), lambda b, t: (b, 0, t, 0)),
            pl.BlockSpec((1, 4, T2, 4), lambda b, t: (b, 0, t, 0)),
            pl.BlockSpec((1, 4, T2, HID), lambda b, t: (b, 0, t, 0)),
            full((256, HID)), full((1, HID)),
            full((HID, 256)), full((1, 256)),
        ],
        out_specs=[
            pl.BlockSpec((1, 4, T2, HID), lambda b, t: (b, 0, t, 0)),
            pl.BlockSpec((1, 4, T2, 256), lambda b, t: (b, 0, t, 0)),
        ],
        out_shape=[
            jax.ShapeDtypeStruct((B, 4, 1024, HID), F32),
            jax.ShapeDtypeStruct((B, 4, 1024, 256), F32),
        ],
    )(att4, gl4, af4, wu1, bu1, wau, bau)


_NC, _NS = 2, 16
_NW = _NC * _NS
_G = B * E
_PW = _G // _NW
_C = 256
_NCHUNK = _PW // _C

_sc_mesh = plsc.VectorSubcoreMesh(core_axis_name="c", subcore_axis_name="s",
                                  num_cores=_NC, num_subcores=_NS)


def _sc_gather_body(tab_hbm, idx_hbm, out_hbm, idx_v, rows_v, sem):
    wid = lax.axis_index("s") * _NC + lax.axis_index("c")
    base = wid * _PW

    def chunk(i, carry):
        off = base + i * _C
        pltpu.sync_copy(idx_hbm.at[pl.ds(off, _C)], idx_v)
        pltpu.async_copy(tab_hbm.at[idx_v], rows_v, sem).wait()
        pltpu.sync_copy(rows_v, out_hbm.at[pl.ds(off, _C)])
        return carry

    lax.fori_loop(0, _NCHUNK, chunk, 0)


_sc_gather = pl.kernel(
    _sc_gather_body,
    out_type=jax.ShapeDtypeStruct((_G, 256), F32),
    mesh=_sc_mesh,
    scratch_types=[
        pltpu.VMEM((_C,), jnp.int32),
        pltpu.VMEM((_C, 256), F32),
        pltpu.SemaphoreType.DMA,
    ],
)


def _gather_rows(tab_flat, gidx):
    return _sc_gather(tab_flat, gidx)


def kernel(input_atom, input_bond, atom_graph, bond_graph, num_nbs, node_mask,
           W_af, b_af, W_nb, b_nb, W_na, b_na, W_sa, b_sa, W_u2, b_u2, W_u1, b_u1,
           Wq, bq, Wk, bk, Wv, bv, Wam, bam, Wg, bg):
    W_u2a = W_u2[:, :HID]
    W_u2b = W_u2[:, HID:]
    waf = W_af.T
    baf = b_af.reshape(1, HID)
    wau = jnp.concatenate([W_na.T, W_u2a.T], axis=1)
    bau = jnp.concatenate([b_na, b_u2]).reshape(1, 256)
    wbt = jnp.concatenate([W_nb.T, W_u2b.T], axis=1)
    bbt = jnp.concatenate([b_nb, jnp.zeros_like(b_u2)]).reshape(1, 256)
    wsa = W_sa.T
    bsa = b_sa.reshape(1, HID)
    wk = Wk.T
    bk2 = bk.reshape(1, HID)
    wv = Wv.T
    bv2 = bv.reshape(1, HID)
    wam = Wam.T
    bam2 = bam.reshape(1, HID)
    wq = Wq.T
    bq2 = bq.reshape(1, HID)
    wg = Wg.reshape(1, HID)
    bg2 = bg.reshape(1, 1)
    wu1 = W_u1.T
    bu1 = b_u1.reshape(1, HID)

    offs = (jnp.arange(B, dtype=jnp.int32) * N)[:, None, None]
    aidx = (atom_graph[..., 1].astype(jnp.int32) + offs).reshape(B * E)
    bidx = (bond_graph[..., 1].astype(jnp.int32) + offs).reshape(B * E)
    mask = (jnp.arange(MAXNEI, dtype=jnp.int32)[None, None, :]
            < num_nbs[:, :, None].astype(jnp.int32)).astype(F32)
    nm = node_mask.reshape(B, N, 1)

    af, A_tab, B_tab = _run_pre(input_atom, input_bond, waf, baf, wau, bau, wbt, bbt)
    BGt = _gather_rows(B_tab.reshape(B * N, 256), bidx).reshape(B, E, 256)

    lin = jnp.zeros((B, N, HID), F32)
    for d in range(DEPTH):
        AG = _gather_rows(A_tab.reshape(B * N, 256), aidx).reshape(B, E, 256)
        lin, att, gl = _run_k1(af, AG, BGt, mask, nm, lin,
                               wsa, bsa, wk, bk2, wv, bv2, wam, bam2,
                               wq, bq2, wg, bg2)
        if d < DEPTH - 1:
            af4, atab4 = _run_k2(att.reshape(B, 4, 1024, HID),
                                 gl.reshape(B, 4, 1024, 4),
                                 af.reshape(B, 4, 1024, HID),
                                 wu1, bu1, wau, bau)
            af = af4.reshape(B, N, HID)
            A_tab = atab4.reshape(B, N, 256)
    return lin
